# baseline (speedup 1.0000x reference)
import jax
import jax.numpy as jnp
from jax import lax
from jax.experimental import pallas as pl
from jax.experimental.pallas import tpu as pltpu
from jax.experimental.pallas import tpu_sc as plsc

B = 4096
N = 64
NC, NS = 2, 16
NW = NC * NS
BPW = B // NW
MPC = 8
NCH = BPW // MPC
NQ = N // 16


def _sc_body(x_hbm, maskq_hbm, act_hbm, bsel_hbm, out_hbm, p_hbm,
             blk_v, maskq_v, act_v, bsel_v, sat2_v, out_v, sem):
    _ZERO = jnp.zeros((16,), jnp.float32)
    _ONE = jnp.ones((16,), jnp.float32)
    _EPS = jnp.full((16,), 1e-20, jnp.float32)
    _THR = jnp.full((16,), 0.8, jnp.float32)
    wid = lax.axis_index("s") * NC + lax.axis_index("c")
    base = wid * BPW

    pltpu.sync_copy(maskq_hbm, maskq_v)
    pltpu.sync_copy(act_hbm, act_v)
    pltpu.sync_copy(bsel_hbm, bsel_v)

    a = act_v[...]
    probs = a / (a + _EPS)
    iota = lax.iota(jnp.int32, 16)

    def chunk_body(c, carry):
        handles = [
            pltpu.async_copy(
                x_hbm.at[base + c * MPC + m], blk_v.at[m], sem)
            for m in range(MPC)
        ]
        for h in handles:
            h.wait()
        for m in range(MPC):
            accv = _ZERO
            for q in range(NQ):
                didx = iota + q * 16
                d = plsc.load_gather(
                    blk_v, [jnp.full((16,), m, jnp.int32), didx, didx])
                predf = jnp.where(d > _THR, _ONE, _ZERO)
                accv = accv + jnp.abs(predf - maskq_v[q, :])
            nz = plsc.all_reduce_population_count(accv != 0.0)
            sat2_v[c * MPC + m, :] = jnp.where(nz == 0, _ONE, _ZERO)
        return carry

    lax.fori_loop(0, NCH, chunk_body, 0)

    for t in range(BPW // 2):
        sv = plsc.load_gather(sat2_v, [bsel_v[t, :], iota])
        out_v[t, :] = sv * probs

    pltpu.sync_copy(out_v, out_hbm.at[pl.ds(wid * (BPW // 2), BPW // 2)])

    for r in range(BPW // 16):
        out_v[r, :] = _ZERO
    pltpu.sync_copy(out_v.at[pl.ds(0, BPW // 16)],
                    p_hbm.at[pl.ds(wid * (BPW // 16), BPW // 16)])


@jax.jit
def kernel(x, action, mask):
    maskq = mask.astype(jnp.float32).reshape(NQ, 16)
    act2 = jnp.concatenate([action, action])
    bsel = (jnp.arange(16, dtype=jnp.int32)[None, :] // 8
            + 2 * jnp.arange(BPW // 2, dtype=jnp.int32)[:, None])

    mesh = plsc.VectorSubcoreMesh(
        core_axis_name="c", subcore_axis_name="s",
        num_cores=NC, num_subcores=NS)
    kfn = pl.kernel(
        _sc_body,
        out_type=(
            jax.ShapeDtypeStruct((B // 2, 16), jnp.float32),
            jax.ShapeDtypeStruct((B // 16, 16), jnp.float32),
        ),
        mesh=mesh,
        compiler_params=pltpu.CompilerParams(needs_layout_passes=False),
        scratch_types=[
            pltpu.VMEM((MPC, N, N), jnp.float32),
            pltpu.VMEM((NQ, 16), jnp.float32),
            pltpu.VMEM((16,), jnp.float32),
            pltpu.VMEM((BPW // 2, 16), jnp.int32),
            pltpu.VMEM((BPW, 16), jnp.float32),
            pltpu.VMEM((BPW // 2, 16), jnp.float32),
            pltpu.SemaphoreType.DMA,
        ],
    )
    out, pz = kfn(x, maskq, act2, bsel)
    return out.reshape(B, 8), pz.reshape(1, B)
